# baseline (speedup 1.0000x reference)
import numpy as np

import jax
import jax.numpy as jnp
from jax.experimental import pallas as pl
from jax.experimental.pallas import tpu as pltpu

_U = 3
_D = 131
_N = 128
_L = 2
_SQRT_THRESH = float(np.sqrt((10.0 ** (10.0 / 10.0)) / 1000.0))

_SEQ = ["in0", "in1"]
for _l in range(_L):
    _SEQ.append("l%d_a0_nn" % _l)
    _SEQ.append("l%d_a0_comb" % _l)
    for _i in (1, 2, 3):
        _SEQ.append("l%d_a%d_nn" % (_l, _i))
        _SEQ.append("l%d_a%d_comb" % (_l, _i))
_ALL = _SEQ + ["irs"]

_NB12 = 4
_NB3 = 3
_NSEQ = len(_SEQ)
_NVMEM = 1 + _NSEQ + 4 + 2 + 3


def _body(*refs):
    y_ref = refs[0]
    w0_refs = refs[1:1 + _NSEQ]
    irs_w = refs[1 + _NSEQ:5 + _NSEQ]
    lin_refs = refs[5 + _NSEQ:7 + _NSEQ]
    bh_ref, b3_ref, blin_ref = refs[7 + _NSEQ:10 + _NSEQ]
    w12_refs = refs[_NVMEM:_NVMEM + 2 * _NSEQ]
    w3_refs = refs[_NVMEM + 2 * _NSEQ:_NVMEM + 3 * _NSEQ]
    out_ref = refs[_NVMEM + 3 * _NSEQ]
    wbuf, w3buf, semw, sem3 = refs[_NVMEM + 3 * _NSEQ + 1:]

    def cp12(t):
        return pltpu.make_async_copy(w12_refs[t], wbuf.at[t % _NB12],
                                     semw.at[t % _NB12])

    def cp3(t):
        return pltpu.make_async_copy(w3_refs[t], w3buf.at[t % _NB3],
                                     sem3.at[t % _NB3])

    for t in range(_NB12):
        cp12(t).start()
    for t in range(_NB3):
        cp3(t).start()

    def mm(x, w):
        return jnp.dot(x, w, preferred_element_type=jnp.float32)

    def relu(x):
        return jnp.maximum(x, 0.0)

    def bias_h(d, j):
        return bh_ref[3 * d + j:3 * d + j + 1, :]

    def dnn_stream(k, x):
        h = relu(mm(x, w0_refs[k][...]) + bias_h(k, 0))
        for j, t in ((1, 2 * k), (2, 2 * k + 1)):
            cp12(t).wait()
            h = relu(mm(h, wbuf[t % _NB12]) + bias_h(k, j))
            if t + _NB12 < 2 * _NSEQ:
                cp12(t + _NB12).start()
        cp3(k).wait()
        h = relu(mm(h, w3buf[k % _NB3]) + b3_ref[k:k + 1, :])
        if k + _NB3 < _NSEQ:
            cp3(k + _NB3).start()
        return h

    def dnn_irs(x):
        d = _NSEQ
        h = relu(mm(x, irs_w[0][...]) + bias_h(d, 0))
        h = relu(mm(h, irs_w[1][...]) + bias_h(d, 1))
        h = relu(mm(h, irs_w[2][...]) + bias_h(d, 2))
        h = relu(mm(h, irs_w[3][...]) + b3_ref[d:d + 1, :])
        return h

    Yv = y_ref[...]
    avg0 = jnp.mean(Yv, axis=0, keepdims=True)
    A = dnn_stream(0, avg0)
    Yc = dnn_stream(1, Yv)
    k = 2
    for _ in range(_L):
        neigh = dnn_stream(k, Yc)
        k += 1
        agg = jnp.mean(neigh, axis=0, keepdims=True)
        irs1 = dnn_irs(A)
        A = dnn_stream(k, jnp.concatenate([irs1, agg], axis=1))
        k += 1
        irs2 = dnn_irs(A)
        temp = Yc
        rows = []
        for i in (1, 2, 3):
            parts = [temp[:i]] + ([temp[i + 1:]] if i < _U else [])
            nb = jnp.concatenate(parts, axis=0) if len(parts) > 1 else parts[0]
            nh = dnn_stream(k, nb)
            k += 1
            aggi = jnp.max(nh, axis=0, keepdims=True)
            mid = jnp.concatenate([irs2, temp[i:i + 1], aggi], axis=1)
            rows.append(dnn_stream(k, mid))
            k += 1
        Yc = jnp.concatenate(rows + [temp[_U:_U + 1]], axis=0)

    v0 = mm(A, lin_refs[0][...]) + blin_ref[0:1, :]
    W0 = mm(Yc, lin_refs[1][...]) + blin_ref[1:2, :]
    Wn = W0 / jnp.sqrt(jnp.sum(W0 * W0, axis=1, keepdims=True)) * _SQRT_THRESH
    a = v0[:, :_N]
    b = v0[:, _N:]
    nrm = jnp.sqrt(a * a + b * b)
    v = jnp.concatenate([a / nrm, b / nrm], axis=1)
    out_ref[...] = jnp.concatenate([v, Wn], axis=0)


def _build(interpret=False):
    vm = pl.BlockSpec(memory_space=pltpu.MemorySpace.VMEM)
    hbm = pl.BlockSpec(memory_space=pltpu.MemorySpace.HBM)
    return pl.pallas_call(
        _body,
        out_shape=jax.ShapeDtypeStruct((_U + 2, 2 * _N), jnp.float32),
        in_specs=[vm] * _NVMEM + [hbm] * (3 * _NSEQ),
        out_specs=vm,
        scratch_shapes=[
            pltpu.VMEM((_NB12, 1024, 1024), jnp.float32),
            pltpu.VMEM((_NB3, 1024, _D), jnp.float32),
            pltpu.SemaphoreType.DMA((_NB12,)),
            pltpu.SemaphoreType.DMA((_NB3,)),
        ],
        compiler_params=pltpu.CompilerParams(
            vmem_limit_bytes=100 * 1024 * 1024,
        ),
        interpret=interpret,
    )


def _prep(Y, params):
    p = params
    w0s = [p[n]["w0"] for n in _SEQ]
    irs_w = [p["irs"]["w%d" % j] for j in range(4)]
    lins = [p["lin0_w"], p["lin1_w"]]
    bh = jnp.concatenate(
        [p[n]["b%d" % j][None, :] for n in _ALL for j in range(3)], axis=0)
    b3 = jnp.concatenate([p[n]["b3"][None, :] for n in _ALL], axis=0)
    blin = jnp.concatenate([p["lin0_b"][None, :], p["lin1_b"][None, :]],
                           axis=0)
    w12 = []
    for n in _SEQ:
        w12 += [p[n]["w1"], p[n]["w2"]]
    w3 = [p[n]["w3"] for n in _SEQ]
    return [Y] + w0s + irs_w + lins + [bh, b3, blin] + w12 + w3


def kernel(Y, params):
    return _build()(*_prep(Y, params))
